# baseline (speedup 1.0000x reference)
import functools

import jax
import jax.numpy as jnp
from jax import lax
from jax.experimental import pallas as pl
from jax.experimental.pallas import tpu as pltpu

_BIG = 1e9


def _mine_batch(iou, pst, labels_ref, b, jio, n_classes, i_t):
    R = iou.shape[0]
    I = jnp.zeros((R, 1), dtype=jnp.float32)
    cls = jnp.zeros((R, 1), dtype=jnp.float32)
    tgt = jnp.full((R, 1), float(n_classes), dtype=jnp.float32)
    keys = []
    for c in range(n_classes):
        psc = pst[c:c + 1, :]
        sel = psc > 0.5
        cnt = jnp.sum(sel.astype(jnp.float32))
        pmax = jnp.max(psc)
        jfb = jnp.min(jnp.where(psc == pmax, jio, _BIG))
        fb = (jio == jfb).astype(jnp.float32)
        self_ = jnp.where(cnt <= 1.0, fb, sel.astype(jnp.float32))
        selb = self_ > 0.0
        keys.append(jnp.where(selb, jio * 16384.0 + jnp.floor(psc * 16384.0),
                              _BIG))
        bv = jnp.max(jnp.where(selb, iou, -1.0), axis=1, keepdims=True)
        lab_ok = labels_ref[b, c] != 0
        upd = jnp.logical_and(bv > I, lab_ok)
        strong = jnp.logical_and(upd, bv > i_t)
        I = jnp.where(upd, bv, I)
        cls = jnp.where(upd, float(c), cls)
        tgt = jnp.where(jnp.logical_and(strong, tgt == float(n_classes)),
                        float(c), tgt)
    keyw = jnp.where(cls == 0.0, keys[0],
                     jnp.where(cls == 1.0, keys[1], keys[2]))
    kwin = jnp.min(jnp.where(iou == I, keyw, _BIG),
                   axis=1, keepdims=True)
    wq = kwin - jnp.floor(kwin * (1.0 / 16384.0)) * 16384.0
    wv = jnp.where(I > 0.0, wq * (1.0 / 16384.0), 0.0)
    return wv, tgt


def _icr_body(x_ref, rois_ref, ps_ref, labels_ref, scale_ref, fcw_ref,
              fcb_ref, xr_ref, loss_ref, *, bs, n_classes, i_t, n_total):
    wsum = jnp.float32(0.0)
    fsum = jnp.float32(0.0)
    for b in range(bs):
        rois = rois_ref[b]
        roist = rois_ref[b].T
        pst = ps_ref[b].T

        rx1 = rois[:, 0:1]
        ry1 = rois[:, 1:2]
        rx2 = rois[:, 2:3]
        ry2 = rois[:, 3:4]
        cx1 = roist[0:1, :]
        cy1 = roist[1:2, :]
        cx2 = roist[2:3, :]
        cy2 = roist[3:4, :]
        area_r = (rx2 - rx1) * (ry2 - ry1)
        area_c = (cx2 - cx1) * (cy2 - cy1)

        ltx = jnp.maximum(rx1, cx1)
        lty = jnp.maximum(ry1, cy1)
        rbx = jnp.minimum(rx2, cx2)
        rby = jnp.minimum(ry2, cy2)
        iw = jnp.maximum(rbx - ltx, 0.0)
        ih = jnp.maximum(rby - lty, 0.0)
        inter = iw * ih
        iou = inter / (area_r + area_c - inter)

        P = iou.shape[1]
        jio = lax.broadcasted_iota(jnp.int32, (1, P), 1).astype(jnp.float32)
        wv, tgt = _mine_batch(iou, pst, labels_ref, b, jio, n_classes, i_t)

        x = x_ref[pl.ds(b * P, P), :]
        wmat = fcw_ref[...]
        logits = lax.dot_general(x, wmat, (((1,), (1,)), ((), ())),
                                 preferred_element_type=jnp.float32)
        logits = logits + fcb_ref[...]
        m1 = jnp.max(logits, axis=1, keepdims=True)
        e1 = jnp.exp(logits - m1)
        xr = e1 / jnp.sum(e1, axis=1, keepdims=True)
        xr_ref[b] = xr

        m2 = jnp.max(xr, axis=1, keepdims=True)
        e2 = jnp.exp(xr - m2)
        p = e2 / jnp.sum(e2, axis=1, keepdims=True)
        eps = 1e-07
        p = jnp.clip(p, eps, 1.0 - eps)
        K = xr.shape[1]
        cio = lax.broadcasted_iota(jnp.int32, (1, K), 1).astype(jnp.float32)
        pt = jnp.sum(jnp.where(cio == tgt, p, 0.0), axis=1, keepdims=True)
        focal = -jnp.log(pt) * (1.0 - pt) ** 2

        wsum = wsum + jnp.sum(wv)
        fsum = fsum + jnp.sum(focal)

    loss_ref[0, 0] = (wsum * fsum * (1.0 / float(n_total * n_total))
                      * scale_ref[0, 0].astype(jnp.float32))


def kernel(inputs, pre_score, labels, rois, num, fc_w, fc_b):
    bs, P, C = pre_score.shape
    K, D = fc_w.shape
    N = bs * P

    labels32 = labels.astype(jnp.int32)
    fcb2 = fc_b.reshape(1, K)
    scale = jnp.asarray(num, jnp.int32).reshape(1, 1) // jnp.int32(P)

    out_shapes = (
        jax.ShapeDtypeStruct((bs, P, K), jnp.float32),
        jax.ShapeDtypeStruct((1, 1), jnp.float32),
    )
    in_specs = [
        pl.BlockSpec((N, D), lambda: (0, 0)),
        pl.BlockSpec((bs, P, 4), lambda: (0, 0, 0)),
        pl.BlockSpec((bs, P, C), lambda: (0, 0, 0)),
        pl.BlockSpec(memory_space=pltpu.SMEM),
        pl.BlockSpec(memory_space=pltpu.SMEM),
        pl.BlockSpec((K, D), lambda: (0, 0)),
        pl.BlockSpec((1, K), lambda: (0, 0)),
    ]
    out_specs = (
        pl.BlockSpec((bs, P, K), lambda: (0, 0, 0)),
        pl.BlockSpec((1, 1), lambda: (0, 0), memory_space=pltpu.SMEM),
    )
    body = functools.partial(_icr_body, bs=bs, n_classes=C, i_t=0.5,
                             n_total=N)
    xr_k, loss = pl.pallas_call(
        body,
        in_specs=in_specs,
        out_specs=out_specs,
        out_shape=out_shapes,
    )(inputs, rois, pre_score, labels32, scale, fc_w, fcb2)

    return (xr_k, loss[0, 0])
